# baseline (device time: 8662 ns/iter reference)
import jax
import jax.numpy as jnp
from jax import lax
from jax.experimental import pallas as pl
from jax.experimental.pallas import tpu as pltpu

N_DEV = 8
BLOCK_M = 512


def kernel(x):
    m_per, n = x.shape
    assert m_per % BLOCK_M == 0
    n_steps = m_per // BLOCK_M

    def body(x_ref, out_ref, local_ref, comm_ref, send_sems, recv_sems):
        my = lax.axis_index("i")
        step = pl.program_id(0)
        barrier_sem = pltpu.get_barrier_semaphore()

        @pl.when(step == 0)
        def _():
            for p in range(N_DEV):
                @pl.when(p != my)
                def _():
                    pl.semaphore_signal(
                        barrier_sem, inc=1,
                        device_id=(p,), device_id_type=pl.DeviceIdType.MESH,
                    )

        partial = jnp.sum(x_ref[...], axis=0, keepdims=True)

        @pl.when(step == 0)
        def _():
            local_ref[...] = partial

        @pl.when(step != 0)
        def _():
            local_ref[...] = local_ref[...] + partial

        @pl.when(step == n_steps - 1)
        def _():
            pl.semaphore_wait(barrier_sem, N_DEV - 1)

            for mask in (6, 2, 5, 7, 1, 3, 4):
                for p in range(N_DEV):
                    @pl.when(p == (my ^ mask))
                    def _():
                        send = pltpu.make_async_remote_copy(
                            src_ref=local_ref,
                            dst_ref=comm_ref.at[pl.ds(my, 1)],
                            send_sem=send_sems.at[p],
                            recv_sem=recv_sems.at[my],
                            device_id=(p,),
                            device_id_type=pl.DeviceIdType.MESH,
                        )
                        send.start()

            out_ref[...] = local_ref[...]

            for mask in (1, 3, 4, 2, 5, 7, 6):
                for p in range(N_DEV):
                    @pl.when(p == (my ^ mask))
                    def _():
                        recv = pltpu.make_async_remote_copy(
                            src_ref=local_ref,
                            dst_ref=comm_ref.at[pl.ds(p, 1)],
                            send_sem=send_sems.at[p],
                            recv_sem=recv_sems.at[p],
                            device_id=(p,),
                            device_id_type=pl.DeviceIdType.MESH,
                        )
                        recv.wait_recv()
                        out_ref[...] = out_ref[...] + comm_ref[pl.ds(p, 1), :]

            for p in range(N_DEV):
                @pl.when(p != my)
                def _():
                    send = pltpu.make_async_remote_copy(
                        src_ref=local_ref,
                        dst_ref=comm_ref.at[pl.ds(p, 1)],
                        send_sem=send_sems.at[p],
                        recv_sem=recv_sems.at[p],
                        device_id=(p,),
                        device_id_type=pl.DeviceIdType.MESH,
                    )
                    send.wait_send()

    return pl.pallas_call(
        body,
        grid=(n_steps,),
        out_shape=jax.ShapeDtypeStruct((1, n), jnp.float32),
        in_specs=[pl.BlockSpec((BLOCK_M, n), lambda i: (i, 0))],
        out_specs=pl.BlockSpec((1, n), lambda i: (0, 0)),
        scratch_shapes=[
            pltpu.VMEM((1, n), jnp.float32),
            pltpu.VMEM((N_DEV, n), jnp.float32),
            pltpu.SemaphoreType.DMA((N_DEV,)),
            pltpu.SemaphoreType.DMA((N_DEV,)),
        ],
        compiler_params=pltpu.CompilerParams(collective_id=0),
    )(x)


# device time: 8551 ns/iter; 1.0130x vs baseline; 1.0130x over previous
import jax
import jax.numpy as jnp
from jax import lax
from jax.experimental import pallas as pl
from jax.experimental.pallas import tpu as pltpu

N_DEV = 8
BLOCK_M = 256


def kernel(x):
    m_per, n = x.shape
    assert m_per % BLOCK_M == 0
    n_steps = m_per // BLOCK_M

    def body(x_ref, out_ref, local_ref, comm_ref, send_sems, recv_sems):
        my = lax.axis_index("i")
        step = pl.program_id(0)
        barrier_sem = pltpu.get_barrier_semaphore()

        @pl.when(step == 0)
        def _():
            for p in range(N_DEV):
                @pl.when(p != my)
                def _():
                    pl.semaphore_signal(
                        barrier_sem, inc=1,
                        device_id=(p,), device_id_type=pl.DeviceIdType.MESH,
                    )

        partial = jnp.sum(x_ref[...], axis=0, keepdims=True)

        @pl.when(step == 0)
        def _():
            local_ref[...] = partial

        @pl.when(step != 0)
        def _():
            local_ref[...] = local_ref[...] + partial

        @pl.when(step == n_steps - 1)
        def _():
            pl.semaphore_wait(barrier_sem, N_DEV - 1)

            for mask in (6, 2, 5, 7, 1, 3, 4):
                for p in range(N_DEV):
                    @pl.when(p == (my ^ mask))
                    def _():
                        send = pltpu.make_async_remote_copy(
                            src_ref=local_ref,
                            dst_ref=comm_ref.at[pl.ds(my, 1)],
                            send_sem=send_sems.at[p],
                            recv_sem=recv_sems.at[my],
                            device_id=(p,),
                            device_id_type=pl.DeviceIdType.MESH,
                        )
                        send.start()

            comm_ref[pl.ds(my, 1), :] = local_ref[...]

            for p in range(N_DEV):
                @pl.when(p != my)
                def _():
                    recv = pltpu.make_async_remote_copy(
                        src_ref=local_ref,
                        dst_ref=comm_ref.at[pl.ds(p, 1)],
                        send_sem=send_sems.at[p],
                        recv_sem=recv_sems.at[p],
                        device_id=(p,),
                        device_id_type=pl.DeviceIdType.MESH,
                    )
                    recv.wait_recv()

            out_ref[...] = jnp.sum(comm_ref[...], axis=0, keepdims=True)

            for p in range(N_DEV):
                @pl.when(p != my)
                def _():
                    send = pltpu.make_async_remote_copy(
                        src_ref=local_ref,
                        dst_ref=comm_ref.at[pl.ds(p, 1)],
                        send_sem=send_sems.at[p],
                        recv_sem=recv_sems.at[p],
                        device_id=(p,),
                        device_id_type=pl.DeviceIdType.MESH,
                    )
                    send.wait_send()

    return pl.pallas_call(
        body,
        grid=(n_steps,),
        out_shape=jax.ShapeDtypeStruct((1, n), jnp.float32),
        in_specs=[pl.BlockSpec((BLOCK_M, n), lambda i: (i, 0))],
        out_specs=pl.BlockSpec((1, n), lambda i: (0, 0)),
        scratch_shapes=[
            pltpu.VMEM((1, n), jnp.float32),
            pltpu.VMEM((N_DEV, n), jnp.float32),
            pltpu.SemaphoreType.DMA((N_DEV,)),
            pltpu.SemaphoreType.DMA((N_DEV,)),
        ],
        compiler_params=pltpu.CompilerParams(collective_id=0),
    )(x)
